# baseline (speedup 1.0000x reference)
import jax
import jax.numpy as jnp
from jax.experimental import pallas as pl

_B, _C, _D = 16384, 26, 8
_R = _C * _D
_BLK = 2048
_GRID = _B // _BLK


def _body(pat_ref, out_ref):
    out_ref[...] = jnp.broadcast_to(pat_ref[...], (_R, _BLK))


def _tc_broadcast(pat):
    return pl.pallas_call(
        _body,
        grid=(_GRID,),
        in_specs=[pl.BlockSpec((_R, 1), lambda i: (0, 0))],
        out_specs=pl.BlockSpec((_R, _BLK), lambda i: (0, i)),
        out_shape=jax.ShapeDtypeStruct((_R, _B), jnp.float32),
    )(pat)


def kernel(input, table):
    del input
    pat = jnp.tile(table.reshape(-1), _C)[:, None]
    out2d = _tc_broadcast(pat)
    return out2d.reshape(_C, _D, _B).transpose(2, 0, 1)


# speedup vs baseline: 1.0928x; 1.0928x over previous
import jax
import jax.numpy as jnp
from jax.experimental import pallas as pl

_B, _C, _D = 16384, 26, 8
_R = _C * _D
_BLK = 8192
_GRID = _B // _BLK


def _body(pat_ref, out_ref):
    out_ref[...] = jnp.broadcast_to(pat_ref[...], (_R, _BLK))


def _tc_broadcast(pat):
    return pl.pallas_call(
        _body,
        grid=(_GRID,),
        in_specs=[pl.BlockSpec((_R, 1), lambda i: (0, 0))],
        out_specs=pl.BlockSpec((_R, _BLK), lambda i: (0, i)),
        out_shape=jax.ShapeDtypeStruct((_R, _B), jnp.float32),
    )(pat)


def kernel(input, table):
    del input
    pat = jnp.tile(table.reshape(-1), _C)[:, None]
    out2d = _tc_broadcast(pat)
    return out2d.reshape(_C, _D, _B).transpose(2, 0, 1)


# speedup vs baseline: 1.1237x; 1.0283x over previous
import jax
import jax.numpy as jnp
from jax.experimental import pallas as pl

_B, _C, _D = 16384, 26, 8
_R = _C * _D
_BLK = 4096
_GRID = _B // _BLK


def _body(pat_ref, out_ref):
    out_ref[...] = jnp.broadcast_to(pat_ref[...], (_R, _BLK))


def _tc_broadcast(pat):
    return pl.pallas_call(
        _body,
        grid=(_GRID,),
        in_specs=[pl.BlockSpec((_R, 1), lambda i: (0, 0))],
        out_specs=pl.BlockSpec((_R, _BLK), lambda i: (0, i)),
        out_shape=jax.ShapeDtypeStruct((_R, _B), jnp.float32),
    )(pat)


def kernel(input, table):
    del input
    pat = jnp.tile(table.reshape(-1), _C)[:, None]
    out2d = _tc_broadcast(pat)
    return out2d.reshape(_C, _D, _B).transpose(2, 0, 1)


# speedup vs baseline: 1.4642x; 1.3030x over previous
import jax
import jax.numpy as jnp
from jax.experimental import pallas as pl
from jax.experimental.pallas import tpu as pltpu

_B, _C, _D = 16384, 26, 8
_R = _C * _D
_BLK = 4096
_GRID = _B // _BLK


def _body(tab_ref, out_ref, pat_ref):
    @pl.when(pl.program_id(0) == 0)
    def _build():
        rid = jax.lax.broadcasted_iota(jnp.int32, (_R, 128), 0)
        r8 = jax.lax.rem(rid, _D)
        acc = jnp.full((_R, 128), tab_ref[0, 0], jnp.float32)
        for d in range(1, _D):
            acc = jnp.where(r8 == d, tab_ref[0, d], acc)
        pat_ref[...] = acc

    out_ref[...] = jnp.tile(pat_ref[...], (1, _BLK // 128))


def _tc_broadcast(table):
    return pl.pallas_call(
        _body,
        grid=(_GRID,),
        in_specs=[pl.BlockSpec(memory_space=pltpu.SMEM)],
        out_specs=pl.BlockSpec((_R, _BLK), lambda i: (0, i)),
        out_shape=jax.ShapeDtypeStruct((_R, _B), jnp.float32),
        scratch_shapes=[pltpu.VMEM((_R, 128), jnp.float32)],
    )(table)


def kernel(input, table):
    del input
    out2d = _tc_broadcast(table)
    return out2d.reshape(_C, _D, _B).transpose(2, 0, 1)
